# baseline (speedup 1.0000x reference)
import jax
import jax.numpy as jnp
from jax import lax
from jax.experimental import pallas as pl
from jax.experimental.pallas import tpu as pltpu
from jax.experimental.pallas import tpu_sc as plsc

NUM_CORES = 2
NUM_SUBCORES = 16
NW = NUM_CORES * NUM_SUBCORES

BATCH = 16384
LEVELS = 7
DIM = 64
VOCAB_TOTAL = 508
ROWS = BATCH * LEVELS
RPW = ROWS // NW
CH = 448
BPC = CH // LEVELS
NCH = RPW // CH


def _idx_body(p_ref, o_ref):
    x = p_ref[...]
    lvl = lax.broadcasted_iota(jnp.int32, (1, LEVELS), 1)
    y = x + (jnp.left_shift(4, lvl) - 4)
    o_ref[...] = lax.pad(y, jnp.int32(0), ((0, 0, 0), (0, 128 - LEVELS, 0)))


def _body(idx_ref, table_ref, out_ref, tbuf, pbufA, pbufB, obufA, obufB,
          lsem, isem, ssem):
    wid = lax.axis_index("s") * NUM_CORES + lax.axis_index("c")
    bw = wid * (BATCH // NW)

    c_t = pltpu.async_copy(table_ref, tbuf, lsem)
    pbufs = [pbufA, pbufB]
    obufs = [obufA, obufB]
    i_copy = {
        0: pltpu.async_copy(idx_ref.at[pl.ds(bw, BPC)], pbufA, isem.at[0])
    }
    c_t.wait()

    iota = lax.iota(jnp.int32, 16)
    sevens = jnp.full((16,), LEVELS, jnp.int32)
    s_copy = {}

    for ch in range(NCH):
        if ch + 1 < NCH:
            i_copy[ch + 1] = pltpu.async_copy(
                idx_ref.at[pl.ds(bw + (ch + 1) * BPC, BPC)],
                pbufs[(ch + 1) % 2], isem.at[(ch + 1) % 2])
        i_copy[ch].wait()
        if ch >= 2:
            s_copy[ch - 2].wait()
        pb = pbufs[ch % 2]
        ob = obufs[ch % 2]

        def fill(i, carry, ch=ch, pb=pb, ob=ob):
            t = ch * CH + i * 16
            brel = lax.div(t + iota, sevens) - ch * BPC
            l16 = lax.rem(t + iota, sevens)
            iv = plsc.load_gather(pb, [brel, l16])
            for j in range(16):
                ivj = lax.gather(
                    iv, jnp.full((16, 1), j, jnp.int32),
                    dimension_numbers=lax.GatherDimensionNumbers(
                        offset_dims=(), collapsed_slice_dims=(0,),
                        start_index_map=(0,)),
                    slice_sizes=(1,),
                    mode=lax.GatherScatterMode.PROMISE_IN_BOUNDS)
                for g in range(DIM // 16):
                    v = plsc.load_gather(tbuf, [ivj, g * 16 + iota])
                    ob[i * 16 + j, pl.ds(g * 16, 16)] = v
            return carry

        lax.fori_loop(0, CH // 16, fill, 0)
        s_copy[ch] = pltpu.async_copy(
            ob, out_ref.at[pl.ds(wid * RPW + ch * CH, CH)],
            ssem.at[ch % 2])

    s_copy[NCH - 2].wait()
    s_copy[NCH - 1].wait()


@jax.jit
def kernel(paths, W0, W1, W2, W3, W4, W5, W6):
    table = jnp.concatenate([W0, W1, W2, W3, W4, W5, W6], axis=0)
    idx128 = pl.pallas_call(
        _idx_body,
        grid=(8,),
        in_specs=[pl.BlockSpec((BATCH // 8, LEVELS), lambda i: (i, 0))],
        out_specs=pl.BlockSpec((BATCH // 8, 128), lambda i: (i, 0)),
        out_shape=jax.ShapeDtypeStruct((BATCH, 128), jnp.int32),
    )(paths)

    mesh = plsc.VectorSubcoreMesh(core_axis_name="c", subcore_axis_name="s")
    out = pl.kernel(
        _body,
        out_type=jax.ShapeDtypeStruct((ROWS, DIM), jnp.float32),
        mesh=mesh,
        compiler_params=pltpu.CompilerParams(
            use_tc_tiling_on_sc=False, needs_layout_passes=False),
        scratch_types=[
            pltpu.VMEM((VOCAB_TOTAL, DIM), jnp.float32),
            pltpu.VMEM((BPC, 128), jnp.int32),
            pltpu.VMEM((BPC, 128), jnp.int32),
            pltpu.VMEM((CH, DIM), jnp.float32),
            pltpu.VMEM((CH, DIM), jnp.float32),
            pltpu.SemaphoreType.DMA,
            pltpu.SemaphoreType.DMA((2,)),
            pltpu.SemaphoreType.DMA((2,)),
        ],
    )(idx128, table)
    return out.reshape(BATCH, LEVELS * DIM)


# speedup vs baseline: 1.0018x; 1.0018x over previous
import jax
import jax.numpy as jnp
from jax import lax
from jax.experimental import pallas as pl
from jax.experimental.pallas import tpu as pltpu
from jax.experimental.pallas import tpu_sc as plsc

NUM_CORES = 2
NUM_SUBCORES = 16
NW = NUM_CORES * NUM_SUBCORES

BATCH = 16384
LEVELS = 7
DIM = 64
VOCAB_TOTAL = 508
ROWS = BATCH * LEVELS
RPW = ROWS // NW
CH = 448
BPC = CH // LEVELS
NCH = RPW // CH


def _idx_body(p_ref, o_ref):
    x = p_ref[...]
    lvl = lax.broadcasted_iota(jnp.int32, (1, LEVELS), 1)
    y = x + (jnp.left_shift(4, lvl) - 4)
    o_ref[...] = lax.pad(y, jnp.int32(0), ((0, 0, 0), (0, 128 - LEVELS, 0)))


def _body(idx_ref, table_ref, out_ref, tbuf, pbufA, pbufB, obufA, obufB,
          lsem, isem, ssem):
    wid = lax.axis_index("s") * NUM_CORES + lax.axis_index("c")
    bw = wid * (BATCH // NW)

    c_t = pltpu.async_copy(table_ref, tbuf, lsem)
    pbufs = [pbufA, pbufB]
    obufs = [obufA, obufB]
    i_copy = {
        0: pltpu.async_copy(idx_ref.at[pl.ds(bw * 128, BPC * 128)], pbufA,
                            isem.at[0])
    }
    c_t.wait()

    iota = lax.iota(jnp.int32, 16)
    sevens = jnp.full((16,), LEVELS, jnp.int32)
    s_copy = {}

    for ch in range(NCH):
        if ch + 1 < NCH:
            i_copy[ch + 1] = pltpu.async_copy(
                idx_ref.at[pl.ds((bw + (ch + 1) * BPC) * 128, BPC * 128)],
                pbufs[(ch + 1) % 2], isem.at[(ch + 1) % 2])
        i_copy[ch].wait()
        if ch >= 2:
            s_copy[ch - 2].wait()
        pb = pbufs[ch % 2]
        ob = obufs[ch % 2]

        def fill(i, carry, ch=ch, pb=pb, ob=ob):
            t = ch * CH + i * 16
            brel = lax.div(t + iota, sevens) - ch * BPC
            l16 = lax.rem(t + iota, sevens)
            iv = plsc.load_gather(pb, [brel * 128 + l16])
            for j in range(16):
                ivj = lax.gather(
                    iv, jnp.full((16, 1), j, jnp.int32),
                    dimension_numbers=lax.GatherDimensionNumbers(
                        offset_dims=(), collapsed_slice_dims=(0,),
                        start_index_map=(0,)),
                    slice_sizes=(1,),
                    mode=lax.GatherScatterMode.PROMISE_IN_BOUNDS)
                for g in range(DIM // 16):
                    v = plsc.load_gather(tbuf, [ivj, g * 16 + iota])
                    ob[i * 16 + j, pl.ds(g * 16, 16)] = v
            return carry

        lax.fori_loop(0, CH // 16, fill, 0)
        s_copy[ch] = pltpu.async_copy(
            ob, out_ref.at[pl.ds(wid * RPW + ch * CH, CH)],
            ssem.at[ch % 2])

    s_copy[NCH - 2].wait()
    s_copy[NCH - 1].wait()


@jax.jit
def kernel(paths, W0, W1, W2, W3, W4, W5, W6):
    table = jnp.concatenate([W0, W1, W2, W3, W4, W5, W6], axis=0)
    idx128 = pl.pallas_call(
        _idx_body,
        grid=(8,),
        in_specs=[pl.BlockSpec((BATCH // 8, LEVELS), lambda i: (i, 0))],
        out_specs=pl.BlockSpec((BATCH // 8, 128), lambda i: (i, 0)),
        out_shape=jax.ShapeDtypeStruct((BATCH, 128), jnp.int32),
    )(paths)
    idx_flat = idx128.reshape(BATCH * 128)

    mesh = plsc.VectorSubcoreMesh(core_axis_name="c", subcore_axis_name="s")
    out = pl.kernel(
        _body,
        out_type=jax.ShapeDtypeStruct((ROWS, DIM), jnp.float32),
        mesh=mesh,
        compiler_params=pltpu.CompilerParams(
            use_tc_tiling_on_sc=False, needs_layout_passes=False),
        scratch_types=[
            pltpu.VMEM((VOCAB_TOTAL, DIM), jnp.float32),
            pltpu.VMEM((BPC * 128,), jnp.int32),
            pltpu.VMEM((BPC * 128,), jnp.int32),
            pltpu.VMEM((CH, DIM), jnp.float32),
            pltpu.VMEM((CH, DIM), jnp.float32),
            pltpu.SemaphoreType.DMA,
            pltpu.SemaphoreType.DMA((2,)),
            pltpu.SemaphoreType.DMA((2,)),
        ],
    )(idx_flat, table)
    return out.reshape(BATCH, LEVELS * DIM)
